# baseline (speedup 1.0000x reference)
import functools

import jax
import jax.numpy as jnp
from jax import lax
from jax.experimental import pallas as pl
from jax.experimental.pallas import tpu as pltpu

_LAMB = 10.0
_NIT = 50
_B, _T, _D = 16, 512, 512
_L = (_T + 1) * (_D + 1)


def _sinkhorn_body(nd_ref, nt_ref, aff_ref, t_ref, a_ref):
    b = pl.program_id(0)
    nd = nd_ref[b]
    nt = nt_ref[b]
    ndf = nd.astype(jnp.float32)
    ntf = nt.astype(jnp.float32)
    aff = aff_ref[...]
    I = lax.broadcasted_iota(jnp.int32, (_T, 1), 0)
    J = lax.broadcasted_iota(jnp.int32, (1, _D), 1)
    core = (I < nt) & (J < nd)
    valid = (I <= nt) & (J <= nd)
    K = jnp.where(valid, jnp.where(core, jnp.exp(_LAMB * aff), 1.0), 0.0)
    r = jnp.where(I < nt, 1.0, jnp.where(I == nt, ntf, 0.0))
    c = jnp.where(J < nd, 1.0, jnp.where(J == nd, ndf, 0.0))

    def body(_, uv):
        u, v = uv
        Kv = lax.dot_general(K, v.reshape(_D, 1), (((1,), (0,)), ((), ())))
        u = jnp.where(Kv > 0, r / jnp.where(Kv > 0, Kv, 1.0), 0.0)
        KTu = lax.dot_general(u.reshape(1, _T), K, (((1,), (0,)), ((), ())))
        v = jnp.where(KTu > 0, c / jnp.where(KTu > 0, KTu, 1.0), 0.0)
        return (u, v)

    u, v = lax.fori_loop(
        0, _NIT, body,
        (jnp.ones((_T, 1), jnp.float32), jnp.ones((1, _D), jnp.float32)))
    t = u * K * v
    t_ref[...] = t
    neg = jnp.where(core, t, -jnp.inf)
    row_max = jnp.max(neg, axis=1, keepdims=True)
    col_max = jnp.max(neg, axis=0, keepdims=True)
    assign_core = core & (neg == row_max) & (neg == col_max)
    death = (I < nt) & jnp.logical_not(jnp.any(assign_core, axis=1, keepdims=True))
    birth = (J < nd) & jnp.logical_not(jnp.any(assign_core, axis=0, keepdims=True))
    assign_full = assign_core | ((J == nd) & death) | ((I == nt) & birth)
    a_ref[...] = assign_full.astype(jnp.float32)


@jax.jit
def _tc_stage(affinity_scores, num_detections, num_tracklets):
    return pl.pallas_call(
        _sinkhorn_body,
        grid=(_B,),
        in_specs=[
            pl.BlockSpec(memory_space=pltpu.SMEM),
            pl.BlockSpec(memory_space=pltpu.SMEM),
            pl.BlockSpec((None, _T, _D), lambda b: (b, 0, 0)),
        ],
        out_specs=[
            pl.BlockSpec((None, _T, _D), lambda b: (b, 0, 0)),
            pl.BlockSpec((None, _T, _D), lambda b: (b, 0, 0)),
        ],
        out_shape=[
            jax.ShapeDtypeStruct((_B, _T, _D), jnp.float32),
            jax.ShapeDtypeStruct((_B, _T, _D), jnp.float32),
        ],
    )(num_detections, num_tracklets, affinity_scores)


def _flatten_one(tb, ab, ndb, ntb):
    nd1 = ndb + 1
    k = jnp.arange(_L, dtype=jnp.int32)
    ii = k // nd1
    jj = k - ii * nd1
    in_range = k < (ntb + 1) * nd1
    iic = jnp.minimum(ii, _T - 1)
    tf = jnp.where(in_range, tb[iic, jj], 0.0)
    af = jnp.where(in_range, ab[iic, jj] != 0.0, False)
    return tf, af


def kernel(affinity_scores, num_detections, num_tracklets):
    t_dense, a_dense = _tc_stage(affinity_scores, num_detections, num_tracklets)
    t_flat, a_flat = jax.vmap(_flatten_one)(
        t_dense, a_dense, num_detections, num_tracklets)
    return t_flat, a_flat


# speedup vs baseline: 1.0765x; 1.0765x over previous
import functools

import jax
import jax.numpy as jnp
from jax import lax
from jax.experimental import pallas as pl
from jax.experimental.pallas import tpu as pltpu

_LAMB = 10.0
_NIT = 50
_B, _T, _D = 16, 512, 512
_L = (_T + 1) * (_D + 1)


def _exp_body(nd_ref, nt_ref, aff_ref, k_ref):
    b = pl.program_id(0)
    nd = nd_ref[b]
    nt = nt_ref[b]
    I = lax.broadcasted_iota(jnp.int32, (_T, 1), 0)
    J = lax.broadcasted_iota(jnp.int32, (1, _D), 1)
    core = (I < nt) & (J < nd)
    valid = (I <= nt) & (J <= nd)
    k_ref[...] = jnp.where(
        valid, jnp.where(core, jnp.exp(_LAMB * aff_ref[...]), 1.0), 0.0)


def _sinkhorn_all(nd_ref, nt_ref, k_ref, t_ref):
    I = lax.broadcasted_iota(jnp.int32, (_T, 1), 0)
    J = lax.broadcasted_iota(jnp.int32, (1, _D), 1)
    rs = []
    cs = []
    for b in range(_B):
        nd = nd_ref[b]
        nt = nt_ref[b]
        ndf = nd.astype(jnp.float32)
        ntf = nt.astype(jnp.float32)
        rs.append(jnp.where(I < nt, 1.0, jnp.where(I == nt, ntf, 0.0)))
        cs.append(jnp.where(J < nd, 1.0, jnp.where(J == nd, ndf, 0.0)))

    def body(_, uv):
        us, vs = uv
        us2 = []
        vs2 = []
        for b in range(_B):
            K = k_ref[b]
            Kv = lax.dot_general(K, vs[b].reshape(_D, 1),
                                 (((1,), (0,)), ((), ())))
            u = jnp.where(Kv > 0, rs[b] / jnp.where(Kv > 0, Kv, 1.0), 0.0)
            KTu = lax.dot_general(u.reshape(1, _T), K,
                                  (((1,), (0,)), ((), ())))
            v = jnp.where(KTu > 0, cs[b] / jnp.where(KTu > 0, KTu, 1.0), 0.0)
            us2.append(u)
            vs2.append(v)
        return (tuple(us2), tuple(vs2))

    u0 = tuple(jnp.ones((_T, 1), jnp.float32) for _ in range(_B))
    v0 = tuple(jnp.ones((1, _D), jnp.float32) for _ in range(_B))
    us, vs = lax.fori_loop(0, _NIT, body, (u0, v0))
    for b in range(_B):
        t_ref[b] = us[b] * k_ref[b] * vs[b]


def _assign_body(nd_ref, nt_ref, t_ref, a_ref):
    b = pl.program_id(0)
    nd = nd_ref[b]
    nt = nt_ref[b]
    I = lax.broadcasted_iota(jnp.int32, (_T, 1), 0)
    J = lax.broadcasted_iota(jnp.int32, (1, _D), 1)
    core = (I < nt) & (J < nd)
    t = t_ref[...]
    neg = jnp.where(core, t, -jnp.inf)
    row_max = jnp.max(neg, axis=1, keepdims=True)
    col_max = jnp.max(neg, axis=0, keepdims=True)
    assign_core = core & (neg == row_max) & (neg == col_max)
    death = (I < nt) & jnp.logical_not(
        jnp.any(assign_core, axis=1, keepdims=True))
    birth = (J < nd) & jnp.logical_not(
        jnp.any(assign_core, axis=0, keepdims=True))
    assign_full = assign_core | ((J == nd) & death) | ((I == nt) & birth)
    a_ref[...] = assign_full.astype(jnp.float32)


@jax.jit
def _tc_stage(affinity_scores, num_detections, num_tracklets):
    k_dense = pl.pallas_call(
        _exp_body,
        grid=(_B,),
        in_specs=[
            pl.BlockSpec(memory_space=pltpu.SMEM),
            pl.BlockSpec(memory_space=pltpu.SMEM),
            pl.BlockSpec((None, _T, _D), lambda b: (b, 0, 0)),
        ],
        out_specs=pl.BlockSpec((None, _T, _D), lambda b: (b, 0, 0)),
        out_shape=jax.ShapeDtypeStruct((_B, _T, _D), jnp.float32),
    )(num_detections, num_tracklets, affinity_scores)
    t_dense = pl.pallas_call(
        _sinkhorn_all,
        in_specs=[
            pl.BlockSpec(memory_space=pltpu.SMEM),
            pl.BlockSpec(memory_space=pltpu.SMEM),
            pl.BlockSpec(memory_space=pltpu.VMEM),
        ],
        out_specs=pl.BlockSpec(memory_space=pltpu.VMEM),
        out_shape=jax.ShapeDtypeStruct((_B, _T, _D), jnp.float32),
    )(num_detections, num_tracklets, k_dense)
    a_dense = pl.pallas_call(
        _assign_body,
        grid=(_B,),
        in_specs=[
            pl.BlockSpec(memory_space=pltpu.SMEM),
            pl.BlockSpec(memory_space=pltpu.SMEM),
            pl.BlockSpec((None, _T, _D), lambda b: (b, 0, 0)),
        ],
        out_specs=pl.BlockSpec((None, _T, _D), lambda b: (b, 0, 0)),
        out_shape=jax.ShapeDtypeStruct((_B, _T, _D), jnp.float32),
    )(num_detections, num_tracklets, t_dense)
    return t_dense, a_dense


def _flatten_one(tb, ab, ndb, ntb):
    nd1 = ndb + 1
    k = jnp.arange(_L, dtype=jnp.int32)
    ii = k // nd1
    jj = k - ii * nd1
    in_range = k < (ntb + 1) * nd1
    iic = jnp.minimum(ii, _T - 1)
    tf = jnp.where(in_range, tb[iic, jj], 0.0)
    af = jnp.where(in_range, ab[iic, jj] != 0.0, False)
    return tf, af


def kernel(affinity_scores, num_detections, num_tracklets):
    t_dense, a_dense = _tc_stage(affinity_scores, num_detections, num_tracklets)
    t_flat, a_flat = jax.vmap(_flatten_one)(
        t_dense, a_dense, num_detections, num_tracklets)
    return t_flat, a_flat


# speedup vs baseline: 1.1013x; 1.0230x over previous
import functools

import jax
import jax.numpy as jnp
from jax import lax
from jax.experimental import pallas as pl
from jax.experimental.pallas import tpu as pltpu

_LAMB = 10.0
_NIT = 50
_B, _T, _D = 16, 512, 512
_L = (_T + 1) * (_D + 1)


def _exp_body(nd_ref, nt_ref, aff_ref, k_ref):
    b = pl.program_id(0)
    nd = nd_ref[b]
    nt = nt_ref[b]
    I = lax.broadcasted_iota(jnp.int32, (_T, 1), 0)
    J = lax.broadcasted_iota(jnp.int32, (1, _D), 1)
    core = (I < nt) & (J < nd)
    valid = (I <= nt) & (J <= nd)
    k_ref[...] = jnp.where(
        valid, jnp.where(core, jnp.exp(_LAMB * aff_ref[...]), 1.0), 0.0)


def _sinkhorn_all(nd_ref, nt_ref, k_ref, t_ref, r_ref, c_ref):
    I = lax.broadcasted_iota(jnp.int32, (_T, 1), 0)
    for b in range(_B):
        nd = nd_ref[b]
        nt = nt_ref[b]
        ndf = nd.astype(jnp.float32)
        ntf = nt.astype(jnp.float32)
        r_ref[:, b:b + 1] = jnp.where(
            I < nt, 1.0, jnp.where(I == nt, ntf, 0.0))
        c_ref[:, b:b + 1] = jnp.where(
            I < nd, 1.0, jnp.where(I == nd, ndf, 0.0))
    R = r_ref[...]
    C = c_ref[...]
    B3 = lax.broadcasted_iota(jnp.int32, (_B, 1, _B), 0)
    L3 = lax.broadcasted_iota(jnp.int32, (_B, 1, _B), 2)
    R_bd = R.reshape(1, _T, _B) * (B3 == L3).astype(jnp.float32)
    K2 = k_ref[...]

    def iterate(V):
        Z = lax.dot_general(K2, V, (((1,), (0,)), ((), ())))
        Z3 = Z.reshape(_B, _T, _B)
        U3 = jnp.where(Z3 > 0, R_bd / jnp.where(Z3 > 0, Z3, 1.0), 0.0)
        U2 = U3.reshape(_B * _T, _B)
        KTu = lax.dot_general(K2, U2, (((0,), (0,)), ((), ())))
        Vn = jnp.where(KTu > 0, C / jnp.where(KTu > 0, KTu, 1.0), 0.0)
        return jnp.sum(U3, axis=0), Vn

    def body(_, uv):
        return iterate(uv[1])

    V0 = jnp.ones((_T, _B), jnp.float32)
    U, V = lax.fori_loop(0, _NIT, body, (V0, V0))
    u3 = jnp.transpose(U).reshape(_B, _T, 1)
    v3 = jnp.transpose(V).reshape(_B, 1, _D)
    t_ref[...] = u3 * K2.reshape(_B, _T, _D) * v3


def _assign_body(nd_ref, nt_ref, t_ref, a_ref):
    b = pl.program_id(0)
    nd = nd_ref[b]
    nt = nt_ref[b]
    I = lax.broadcasted_iota(jnp.int32, (_T, 1), 0)
    J = lax.broadcasted_iota(jnp.int32, (1, _D), 1)
    core = (I < nt) & (J < nd)
    t = t_ref[...]
    neg = jnp.where(core, t, -jnp.inf)
    row_max = jnp.max(neg, axis=1, keepdims=True)
    col_max = jnp.max(neg, axis=0, keepdims=True)
    assign_core = core & (neg == row_max) & (neg == col_max)
    death = (I < nt) & jnp.logical_not(
        jnp.any(assign_core, axis=1, keepdims=True))
    birth = (J < nd) & jnp.logical_not(
        jnp.any(assign_core, axis=0, keepdims=True))
    assign_full = assign_core | ((J == nd) & death) | ((I == nt) & birth)
    a_ref[...] = assign_full.astype(jnp.float32)


@jax.jit
def _tc_stage(affinity_scores, num_detections, num_tracklets):
    k_dense = pl.pallas_call(
        _exp_body,
        grid=(_B,),
        in_specs=[
            pl.BlockSpec(memory_space=pltpu.SMEM),
            pl.BlockSpec(memory_space=pltpu.SMEM),
            pl.BlockSpec((None, _T, _D), lambda b: (b, 0, 0)),
        ],
        out_specs=pl.BlockSpec((None, _T, _D), lambda b: (b, 0, 0)),
        out_shape=jax.ShapeDtypeStruct((_B, _T, _D), jnp.float32),
    )(num_detections, num_tracklets, affinity_scores)
    t_dense = pl.pallas_call(
        _sinkhorn_all,
        in_specs=[
            pl.BlockSpec(memory_space=pltpu.SMEM),
            pl.BlockSpec(memory_space=pltpu.SMEM),
            pl.BlockSpec(memory_space=pltpu.VMEM),
        ],
        out_specs=pl.BlockSpec(memory_space=pltpu.VMEM),
        out_shape=jax.ShapeDtypeStruct((_B, _T, _D), jnp.float32),
        scratch_shapes=[
            pltpu.VMEM((_T, _B), jnp.float32),
            pltpu.VMEM((_T, _B), jnp.float32),
        ],
    )(num_detections, num_tracklets, k_dense.reshape(_B * _T, _D))
    a_dense = pl.pallas_call(
        _assign_body,
        grid=(_B,),
        in_specs=[
            pl.BlockSpec(memory_space=pltpu.SMEM),
            pl.BlockSpec(memory_space=pltpu.SMEM),
            pl.BlockSpec((None, _T, _D), lambda b: (b, 0, 0)),
        ],
        out_specs=pl.BlockSpec((None, _T, _D), lambda b: (b, 0, 0)),
        out_shape=jax.ShapeDtypeStruct((_B, _T, _D), jnp.float32),
    )(num_detections, num_tracklets, t_dense)
    return t_dense, a_dense


def _flatten_one(tb, ab, ndb, ntb):
    nd1 = ndb + 1
    k = jnp.arange(_L, dtype=jnp.int32)
    ii = k // nd1
    jj = k - ii * nd1
    in_range = k < (ntb + 1) * nd1
    iic = jnp.minimum(ii, _T - 1)
    tf = jnp.where(in_range, tb[iic, jj], 0.0)
    af = jnp.where(in_range, ab[iic, jj] != 0.0, False)
    return tf, af


def kernel(affinity_scores, num_detections, num_tracklets):
    t_dense, a_dense = _tc_stage(affinity_scores, num_detections, num_tracklets)
    t_flat, a_flat = jax.vmap(_flatten_one)(
        t_dense, a_dense, num_detections, num_tracklets)
    return t_flat, a_flat
